# Initial kernel scaffold; baseline (speedup 1.0000x reference)
import jax
import jax.numpy as jnp
from jax.experimental import pallas as pl


def kernel(unknown, known, unknow_feats, known_feats, W0, g0, b0, W1, g1, b1):
    raise NotImplementedError("write your pallas kernel here")


# speedup vs baseline: 21.1755x; 21.1755x over previous
import jax
import jax.numpy as jnp
from jax.experimental import pallas as pl
from jax.experimental.pallas import tpu as pltpu

B, N, M, C1, C2 = 4, 8192, 2048, 64, 128
H = 128
EPS_W = 1e-8
EPS_BN = 1e-5
TILE = 256
NT = N // TILE
BIG = 3.0e38
F32 = jnp.float32


def _p1_body(u_ref, k_ref, uf_ref, kf_ref, w0a_ref, w0b_ref,
             h0_ref, s0_ref, q0_ref):
    first = jnp.logical_and(pl.program_id(0) == 0, pl.program_id(1) == 0)

    u = u_ref[0]
    k = k_ref[0]
    u2 = jnp.sum(u * u, axis=1)
    k2 = jnp.sum(k * k, axis=1)
    cross = jax.lax.dot_general(u.astype(jnp.bfloat16), k.astype(jnp.bfloat16),
                                (((1,), (1,)), ((), ())),
                                preferred_element_type=F32)
    d = u2[:, None] + k2[None, :] - 2.0 * cross

    iota = jax.lax.broadcasted_iota(jnp.int32, (TILE, M), 1)
    vals, idxs = [], []
    dc = d
    for _ in range(3):
        m = jnp.min(dc, axis=1, keepdims=True)
        i = jnp.min(jnp.where(dc == m, iota, M), axis=1, keepdims=True)
        vals.append(m)
        idxs.append(i)
        dc = jnp.where(iota == i, BIG, dc)

    w = [1.0 / (v + EPS_W) for v in vals]
    wsum = w[0] + w[1] + w[2]
    w = [wi / wsum for wi in w]

    onehot = (jnp.where(iota == idxs[0], w[0], 0.0)
              + jnp.where(iota == idxs[1], w[1], 0.0)
              + jnp.where(iota == idxs[2], w[2], 0.0))
    interp = jnp.dot(onehot, kf_ref[0], preferred_element_type=F32,
                     precision=jax.lax.Precision.HIGHEST)

    h0 = (jax.lax.dot_general(uf_ref[0], w0a_ref[...], (((0,), (0,)), ((), ())),
                              preferred_element_type=F32)
          + jnp.dot(interp, w0b_ref[...], preferred_element_type=F32))
    h0_ref[0] = h0

    @pl.when(first)
    def _():
        s0_ref[...] = jnp.zeros_like(s0_ref)
        q0_ref[...] = jnp.zeros_like(q0_ref)

    s0_ref[...] += jnp.sum(h0, axis=0, keepdims=True)
    q0_ref[...] += jnp.sum(h0 * h0, axis=0, keepdims=True)


def _p2_body(h0_ref, sc_ref, sh_ref, w1_ref, h1_ref, s1_ref, q1_ref):
    first = jnp.logical_and(pl.program_id(0) == 0, pl.program_id(1) == 0)
    a = jnp.maximum(h0_ref[0] * sc_ref[...] + sh_ref[...], 0.0)
    h1 = jnp.dot(a, w1_ref[...], preferred_element_type=F32)
    h1_ref[0] = h1

    @pl.when(first)
    def _():
        s1_ref[...] = jnp.zeros_like(s1_ref)
        q1_ref[...] = jnp.zeros_like(q1_ref)

    s1_ref[...] += jnp.sum(h1, axis=0, keepdims=True)
    q1_ref[...] += jnp.sum(h1 * h1, axis=0, keepdims=True)


def _p3_body(h1_ref, sc_ref, sh_ref, out_ref):
    y = jnp.maximum(h1_ref[0] * sc_ref[...] + sh_ref[...], 0.0)
    out_ref[0] = y.T


def _affine(s, q, gamma, beta):
    n = float(B * N)
    mean = s / n
    var = q / n - mean * mean
    scale = gamma[None, :] / jnp.sqrt(var + EPS_BN)
    shift = beta[None, :] - mean * scale
    return scale, shift


def kernel(unknown, known, unknow_feats, known_feats, W0, g0, b0, W1, g1, b1):
    kf = jnp.transpose(known_feats, (0, 2, 1))
    w0a = W0[:C1]
    w0b = W0[C1:]

    h0, s0, q0 = pl.pallas_call(
        _p1_body,
        grid=(B, NT),
        in_specs=[
            pl.BlockSpec((1, TILE, 3), lambda b, t: (b, t, 0)),
            pl.BlockSpec((1, M, 3), lambda b, t: (b, 0, 0)),
            pl.BlockSpec((1, C1, TILE), lambda b, t: (b, 0, t)),
            pl.BlockSpec((1, M, C2), lambda b, t: (b, 0, 0)),
            pl.BlockSpec((C1, H), lambda b, t: (0, 0)),
            pl.BlockSpec((C2, H), lambda b, t: (0, 0)),
        ],
        out_specs=[
            pl.BlockSpec((1, TILE, H), lambda b, t: (b, t, 0)),
            pl.BlockSpec((1, H), lambda b, t: (0, 0)),
            pl.BlockSpec((1, H), lambda b, t: (0, 0)),
        ],
        out_shape=[
            jax.ShapeDtypeStruct((B, N, H), F32),
            jax.ShapeDtypeStruct((1, H), F32),
            jax.ShapeDtypeStruct((1, H), F32),
        ],
        compiler_params=pltpu.CompilerParams(
            dimension_semantics=("arbitrary", "arbitrary")),
    )(unknown, known, unknow_feats, kf, w0a, w0b)

    sc0, sh0 = _affine(s0, q0, g0, b0)

    h1, s1, q1 = pl.pallas_call(
        _p2_body,
        grid=(B, NT),
        in_specs=[
            pl.BlockSpec((1, TILE, H), lambda b, t: (b, t, 0)),
            pl.BlockSpec((1, H), lambda b, t: (0, 0)),
            pl.BlockSpec((1, H), lambda b, t: (0, 0)),
            pl.BlockSpec((H, H), lambda b, t: (0, 0)),
        ],
        out_specs=[
            pl.BlockSpec((1, TILE, H), lambda b, t: (b, t, 0)),
            pl.BlockSpec((1, H), lambda b, t: (0, 0)),
            pl.BlockSpec((1, H), lambda b, t: (0, 0)),
        ],
        out_shape=[
            jax.ShapeDtypeStruct((B, N, H), F32),
            jax.ShapeDtypeStruct((1, H), F32),
            jax.ShapeDtypeStruct((1, H), F32),
        ],
        compiler_params=pltpu.CompilerParams(
            dimension_semantics=("arbitrary", "arbitrary")),
    )(h0, sc0, sh0, W1)

    sc1, sh1 = _affine(s1, q1, g1, b1)

    out = pl.pallas_call(
        _p3_body,
        grid=(B, NT),
        in_specs=[
            pl.BlockSpec((1, TILE, H), lambda b, t: (b, t, 0)),
            pl.BlockSpec((1, H), lambda b, t: (0, 0)),
            pl.BlockSpec((1, H), lambda b, t: (0, 0)),
        ],
        out_specs=pl.BlockSpec((1, H, TILE), lambda b, t: (b, 0, t)),
        out_shape=jax.ShapeDtypeStruct((B, H, N), F32),
        compiler_params=pltpu.CompilerParams(
            dimension_semantics=("arbitrary", "arbitrary")),
    )(h1, sc1, sh1)

    return out


# speedup vs baseline: 24.3333x; 1.1491x over previous
import jax
import jax.numpy as jnp
from jax.experimental import pallas as pl
from jax.experimental.pallas import tpu as pltpu

B, N, M, C1, C2 = 4, 8192, 2048, 64, 128
H = 128
EPS_W = 1e-8
EPS_BN = 1e-5
TILE = 512
NT = N // TILE
BIG = 3.0e38
F32 = jnp.float32
BF16 = jnp.bfloat16


def _p1_body(u_ref, k_ref, kt_ref, uf_ref, kf_ref, w0a_ref, w0b_ref,
             h0_ref, s0_ref, q0_ref):
    first = jnp.logical_and(pl.program_id(0) == 0, pl.program_id(1) == 0)

    u = u_ref[0]
    k = k_ref[0]
    kt = kt_ref[0]
    u2 = jnp.sum(u * u, axis=1)
    k2 = (kt[0] * kt[0] + kt[1] * kt[1]) + kt[2] * kt[2]
    cross = jax.lax.dot_general(u.astype(BF16), k.astype(BF16),
                                (((1,), (1,)), ((), ())),
                                preferred_element_type=F32)
    d = u2[:, None] + k2[None, :] - 2.0 * cross

    iota = jax.lax.broadcasted_iota(jnp.int32, (TILE, M), 1)
    dc = d
    wun = []
    oh = None
    for kk in range(3):
        m = jnp.min(dc, axis=1, keepdims=True)
        i = jnp.min(jnp.where(dc == m, iota, M), axis=1, keepdims=True)
        ieq = iota == i
        wk = 1.0 / (m + EPS_W)
        wun.append(wk)
        ohk = jnp.where(ieq, wk, 0.0)
        oh = ohk if oh is None else oh + ohk
        if kk < 2:
            dc = jnp.where(ieq, BIG, dc)

    inv = 1.0 / (wun[0] + wun[1] + wun[2])
    oh = oh * inv
    interp = jnp.dot(oh, kf_ref[0], preferred_element_type=F32,
                     precision=jax.lax.Precision.HIGHEST)

    h0 = (jax.lax.dot_general(uf_ref[0].astype(BF16), w0a_ref[...],
                              (((0,), (0,)), ((), ())),
                              preferred_element_type=F32)
          + jnp.dot(interp.astype(BF16), w0b_ref[...],
                    preferred_element_type=F32))
    h0_ref[0] = h0

    @pl.when(first)
    def _():
        s0_ref[...] = jnp.zeros_like(s0_ref)
        q0_ref[...] = jnp.zeros_like(q0_ref)

    s0_ref[...] += jnp.sum(h0, axis=0, keepdims=True)
    q0_ref[...] += jnp.sum(h0 * h0, axis=0, keepdims=True)


def _p2_body(h0_ref, sc_ref, sh_ref, w1_ref, h1_ref, s1_ref, q1_ref):
    first = jnp.logical_and(pl.program_id(0) == 0, pl.program_id(1) == 0)
    a = jnp.maximum(h0_ref[0] * sc_ref[...] + sh_ref[...], 0.0)
    h1 = jnp.dot(a.astype(BF16), w1_ref[...], preferred_element_type=F32)
    h1_ref[0] = h1

    @pl.when(first)
    def _():
        s1_ref[...] = jnp.zeros_like(s1_ref)
        q1_ref[...] = jnp.zeros_like(q1_ref)

    s1_ref[...] += jnp.sum(h1, axis=0, keepdims=True)
    q1_ref[...] += jnp.sum(h1 * h1, axis=0, keepdims=True)


def _p3_body(h1_ref, sc_ref, sh_ref, out_ref):
    y = jnp.maximum(h1_ref[0] * sc_ref[...] + sh_ref[...], 0.0)
    out_ref[0] = y.T


def _affine(s, q, gamma, beta):
    n = float(B * N)
    mean = s / n
    var = q / n - mean * mean
    scale = gamma[None, :] / jnp.sqrt(var + EPS_BN)
    shift = beta[None, :] - mean * scale
    return scale, shift


def kernel(unknown, known, unknow_feats, known_feats, W0, g0, b0, W1, g1, b1):
    k_t = jnp.transpose(known, (0, 2, 1))
    kf = jnp.transpose(known_feats, (0, 2, 1))
    w0a = W0[:C1].astype(BF16)
    w0b = W0[C1:].astype(BF16)
    w1b = W1.astype(BF16)

    h0, s0, q0 = pl.pallas_call(
        _p1_body,
        grid=(B, NT),
        in_specs=[
            pl.BlockSpec((1, TILE, 3), lambda b, t: (b, t, 0)),
            pl.BlockSpec((1, M, 3), lambda b, t: (b, 0, 0)),
            pl.BlockSpec((1, 3, M), lambda b, t: (b, 0, 0)),
            pl.BlockSpec((1, C1, TILE), lambda b, t: (b, 0, t)),
            pl.BlockSpec((1, M, C2), lambda b, t: (b, 0, 0)),
            pl.BlockSpec((C1, H), lambda b, t: (0, 0)),
            pl.BlockSpec((C2, H), lambda b, t: (0, 0)),
        ],
        out_specs=[
            pl.BlockSpec((1, TILE, H), lambda b, t: (b, t, 0)),
            pl.BlockSpec((1, H), lambda b, t: (0, 0)),
            pl.BlockSpec((1, H), lambda b, t: (0, 0)),
        ],
        out_shape=[
            jax.ShapeDtypeStruct((B, N, H), F32),
            jax.ShapeDtypeStruct((1, H), F32),
            jax.ShapeDtypeStruct((1, H), F32),
        ],
        compiler_params=pltpu.CompilerParams(
            dimension_semantics=("arbitrary", "arbitrary")),
    )(unknown, known, k_t, unknow_feats, kf, w0a, w0b)

    sc0, sh0 = _affine(s0, q0, g0, b0)

    h1, s1, q1 = pl.pallas_call(
        _p2_body,
        grid=(B, NT),
        in_specs=[
            pl.BlockSpec((1, TILE, H), lambda b, t: (b, t, 0)),
            pl.BlockSpec((1, H), lambda b, t: (0, 0)),
            pl.BlockSpec((1, H), lambda b, t: (0, 0)),
            pl.BlockSpec((H, H), lambda b, t: (0, 0)),
        ],
        out_specs=[
            pl.BlockSpec((1, TILE, H), lambda b, t: (b, t, 0)),
            pl.BlockSpec((1, H), lambda b, t: (0, 0)),
            pl.BlockSpec((1, H), lambda b, t: (0, 0)),
        ],
        out_shape=[
            jax.ShapeDtypeStruct((B, N, H), F32),
            jax.ShapeDtypeStruct((1, H), F32),
            jax.ShapeDtypeStruct((1, H), F32),
        ],
        compiler_params=pltpu.CompilerParams(
            dimension_semantics=("arbitrary", "arbitrary")),
    )(h0, sc0, sh0, w1b)

    sc1, sh1 = _affine(s1, q1, g1, b1)

    out = pl.pallas_call(
        _p3_body,
        grid=(B, NT),
        in_specs=[
            pl.BlockSpec((1, TILE, H), lambda b, t: (b, t, 0)),
            pl.BlockSpec((1, H), lambda b, t: (0, 0)),
            pl.BlockSpec((1, H), lambda b, t: (0, 0)),
        ],
        out_specs=pl.BlockSpec((1, H, TILE), lambda b, t: (b, 0, t)),
        out_shape=jax.ShapeDtypeStruct((B, H, N), F32),
        compiler_params=pltpu.CompilerParams(
            dimension_semantics=("arbitrary", "arbitrary")),
    )(h1, sc1, sh1)

    return out


# speedup vs baseline: 34.1530x; 1.4035x over previous
import functools

import jax
import jax.numpy as jnp
from jax import lax
from jax.experimental import pallas as pl
from jax.experimental.pallas import tpu as pltpu
from jax.experimental.pallas import tpu_sc as plsc

B, N, M, C1, C2 = 4, 8192, 2048, 64, 128
H = 128
EPS_W = 1e-8
EPS_BN = 1e-5
TILE = 512
NT = N // TILE
BIG = 3.0e38
F32 = jnp.float32
BF16 = jnp.bfloat16

BN_ = B * N
NW = 32
PPW = BN_ // NW
CH = 128
NCH = PPW // CH


def _p1_body(u_ref, k_ref, kt_ref, idx_ref, w_ref):
    u = u_ref[0]
    k = k_ref[0]
    kt = kt_ref[0]
    u2 = jnp.sum(u * u, axis=1)
    k2 = (kt[0] * kt[0] + kt[1] * kt[1]) + kt[2] * kt[2]
    cross = jax.lax.dot_general(u.astype(BF16), k.astype(BF16),
                                (((1,), (1,)), ((), ())),
                                preferred_element_type=F32)
    d = u2[:, None] + k2[None, :] - 2.0 * cross

    iota = jax.lax.broadcasted_iota(jnp.int32, (TILE, M), 1)
    gbase = pl.program_id(0) * M
    dc = d
    wun, idxs = [], []
    for kk in range(3):
        m = jnp.min(dc, axis=1, keepdims=True)
        i = jnp.min(jnp.where(dc == m, iota, M), axis=1, keepdims=True)
        wun.append(1.0 / (m + EPS_W))
        idxs.append(i)
        if kk < 2:
            dc = jnp.where(iota == i, BIG, dc)

    wsum = (wun[0] + wun[2]) + wun[1]
    for kk in range(3):
        idx_ref[kk] = jnp.transpose(idxs[kk] + gbase, (1, 0))
        w_ref[kk] = jnp.transpose(wun[kk] / wsum, (1, 0))


def _sc_interp(kf_hbm, idx_hbm, w_hbm, out_hbm,
               i0, i1, i2, w0, w1, w2, r0, r1, r2, acc, s0, s1, s2):
    wid = lax.axis_index("s") * 2 + lax.axis_index("c")
    base0 = wid * PPW

    def chunk(c, carry):
        base = base0 + c * CH
        pltpu.sync_copy(idx_hbm.at[0, 0, pl.ds(base, CH)], i0)
        pltpu.sync_copy(idx_hbm.at[1, 0, pl.ds(base, CH)], i1)
        pltpu.sync_copy(idx_hbm.at[2, 0, pl.ds(base, CH)], i2)
        a0 = pltpu.async_copy(kf_hbm.at[i0], r0, s0)
        a1 = pltpu.async_copy(kf_hbm.at[i1], r1, s1)
        a2 = pltpu.async_copy(kf_hbm.at[i2], r2, s2)
        pltpu.sync_copy(w_hbm.at[0, 0, pl.ds(base, CH)], w0)
        pltpu.sync_copy(w_hbm.at[1, 0, pl.ds(base, CH)], w1)
        pltpu.sync_copy(w_hbm.at[2, 0, pl.ds(base, CH)], w2)
        a0.wait()
        a1.wait()
        a2.wait()

        def group(g, carry2):
            wv0 = w0[pl.ds(g * 16, 16)]
            wv1 = w1[pl.ds(g * 16, 16)]
            wv2 = w2[pl.ds(g * 16, 16)]
            for j in range(16):
                p = g * 16 + j
                ws0 = wv0[j]
                ws1 = wv1[j]
                ws2 = wv2[j]
                for v in range(H // 16):
                    sl = pl.ds(v * 16, 16)
                    acc[p, sl] = (r0[p, sl] * ws0 + r1[p, sl] * ws1
                                  + r2[p, sl] * ws2)
            return carry2

        lax.fori_loop(0, CH // 16, group, 0)
        pltpu.sync_copy(acc, out_hbm.at[pl.ds(base, CH)])
        return carry

    lax.fori_loop(0, NCH, chunk, 0)


def _p2_body(uf_ref, in_ref, w0a_ref, w0b_ref, h0_ref, s0_ref, q0_ref):
    first = jnp.logical_and(pl.program_id(0) == 0, pl.program_id(1) == 0)
    h0 = (jax.lax.dot_general(uf_ref[0].astype(BF16), w0a_ref[...],
                              (((0,), (0,)), ((), ())),
                              preferred_element_type=F32)
          + jnp.dot(in_ref[0].astype(BF16), w0b_ref[...],
                    preferred_element_type=F32))
    h0_ref[0] = h0

    @pl.when(first)
    def _():
        s0_ref[...] = jnp.zeros_like(s0_ref)
        q0_ref[...] = jnp.zeros_like(q0_ref)

    s0_ref[...] += jnp.sum(h0, axis=0, keepdims=True)
    q0_ref[...] += jnp.sum(h0 * h0, axis=0, keepdims=True)


def _p3_body(h0_ref, sc_ref, sh_ref, w1_ref, h1_ref, s1_ref, q1_ref):
    first = jnp.logical_and(pl.program_id(0) == 0, pl.program_id(1) == 0)
    a = jnp.maximum(h0_ref[0] * sc_ref[...] + sh_ref[...], 0.0)
    h1 = jnp.dot(a.astype(BF16), w1_ref[...], preferred_element_type=F32)
    h1_ref[0] = h1

    @pl.when(first)
    def _():
        s1_ref[...] = jnp.zeros_like(s1_ref)
        q1_ref[...] = jnp.zeros_like(q1_ref)

    s1_ref[...] += jnp.sum(h1, axis=0, keepdims=True)
    q1_ref[...] += jnp.sum(h1 * h1, axis=0, keepdims=True)


def _p4_body(h1_ref, sc_ref, sh_ref, out_ref):
    y = jnp.maximum(h1_ref[0] * sc_ref[...] + sh_ref[...], 0.0)
    out_ref[0] = y.T


def _affine(s, q, gamma, beta):
    n = float(B * N)
    mean = s / n
    var = q / n - mean * mean
    scale = gamma[None, :] / jnp.sqrt(var + EPS_BN)
    shift = beta[None, :] - mean * scale
    return scale, shift


def kernel(unknown, known, unknow_feats, known_feats, W0, g0, b0, W1, g1, b1):
    k_t = jnp.transpose(known, (0, 2, 1))
    kf = jnp.transpose(known_feats, (0, 2, 1))
    kf_flat = kf.reshape(B * M, C2)
    w0a = W0[:C1].astype(BF16)
    w0b = W0[C1:].astype(BF16)
    w1b = W1.astype(BF16)

    idxg, wn = pl.pallas_call(
        _p1_body,
        grid=(B, NT),
        in_specs=[
            pl.BlockSpec((1, TILE, 3), lambda b, t: (b, t, 0)),
            pl.BlockSpec((1, M, 3), lambda b, t: (b, 0, 0)),
            pl.BlockSpec((1, 3, M), lambda b, t: (b, 0, 0)),
        ],
        out_specs=[
            pl.BlockSpec((3, 1, TILE), lambda b, t: (0, 0, b * NT + t)),
            pl.BlockSpec((3, 1, TILE), lambda b, t: (0, 0, b * NT + t)),
        ],
        out_shape=[
            jax.ShapeDtypeStruct((3, 1, BN_), jnp.int32),
            jax.ShapeDtypeStruct((3, 1, BN_), F32),
        ],
        compiler_params=pltpu.CompilerParams(
            dimension_semantics=("arbitrary", "arbitrary")),
    )(unknown, known, k_t)

    sc_fn = functools.partial(
        pl.kernel,
        out_type=jax.ShapeDtypeStruct((BN_, C2), F32),
        mesh=plsc.VectorSubcoreMesh(core_axis_name="c", subcore_axis_name="s"),
        scratch_types=[
            pltpu.VMEM((CH,), jnp.int32),
            pltpu.VMEM((CH,), jnp.int32),
            pltpu.VMEM((CH,), jnp.int32),
            pltpu.VMEM((CH,), F32),
            pltpu.VMEM((CH,), F32),
            pltpu.VMEM((CH,), F32),
            pltpu.VMEM((CH, C2), F32),
            pltpu.VMEM((CH, C2), F32),
            pltpu.VMEM((CH, C2), F32),
            pltpu.VMEM((CH, C2), F32),
            pltpu.SemaphoreType.DMA,
            pltpu.SemaphoreType.DMA,
            pltpu.SemaphoreType.DMA,
        ],
    )(_sc_interp)
    interp = sc_fn(kf_flat, idxg, wn)
    interp = interp.reshape(B, N, C2)

    h0, s0, q0 = pl.pallas_call(
        _p2_body,
        grid=(B, NT),
        in_specs=[
            pl.BlockSpec((1, C1, TILE), lambda b, t: (b, 0, t)),
            pl.BlockSpec((1, TILE, C2), lambda b, t: (b, t, 0)),
            pl.BlockSpec((C1, H), lambda b, t: (0, 0)),
            pl.BlockSpec((C2, H), lambda b, t: (0, 0)),
        ],
        out_specs=[
            pl.BlockSpec((1, TILE, H), lambda b, t: (b, t, 0)),
            pl.BlockSpec((1, H), lambda b, t: (0, 0)),
            pl.BlockSpec((1, H), lambda b, t: (0, 0)),
        ],
        out_shape=[
            jax.ShapeDtypeStruct((B, N, H), F32),
            jax.ShapeDtypeStruct((1, H), F32),
            jax.ShapeDtypeStruct((1, H), F32),
        ],
        compiler_params=pltpu.CompilerParams(
            dimension_semantics=("arbitrary", "arbitrary")),
    )(unknow_feats, interp, w0a, w0b)

    sc0, sh0 = _affine(s0, q0, g0, b0)

    h1, s1, q1 = pl.pallas_call(
        _p3_body,
        grid=(B, NT),
        in_specs=[
            pl.BlockSpec((1, TILE, H), lambda b, t: (b, t, 0)),
            pl.BlockSpec((1, H), lambda b, t: (0, 0)),
            pl.BlockSpec((1, H), lambda b, t: (0, 0)),
            pl.BlockSpec((H, H), lambda b, t: (0, 0)),
        ],
        out_specs=[
            pl.BlockSpec((1, TILE, H), lambda b, t: (b, t, 0)),
            pl.BlockSpec((1, H), lambda b, t: (0, 0)),
            pl.BlockSpec((1, H), lambda b, t: (0, 0)),
        ],
        out_shape=[
            jax.ShapeDtypeStruct((B, N, H), F32),
            jax.ShapeDtypeStruct((1, H), F32),
            jax.ShapeDtypeStruct((1, H), F32),
        ],
        compiler_params=pltpu.CompilerParams(
            dimension_semantics=("arbitrary", "arbitrary")),
    )(h0, sc0, sh0, w1b)

    sc1, sh1 = _affine(s1, q1, g1, b1)

    out = pl.pallas_call(
        _p4_body,
        grid=(B, NT),
        in_specs=[
            pl.BlockSpec((1, TILE, H), lambda b, t: (b, t, 0)),
            pl.BlockSpec((1, H), lambda b, t: (0, 0)),
            pl.BlockSpec((1, H), lambda b, t: (0, 0)),
        ],
        out_specs=pl.BlockSpec((1, H, TILE), lambda b, t: (b, 0, t)),
        out_shape=jax.ShapeDtypeStruct((B, H, N), F32),
        compiler_params=pltpu.CompilerParams(
            dimension_semantics=("arbitrary", "arbitrary")),
    )(h1, sc1, sh1)

    return out


# speedup vs baseline: 36.5699x; 1.0708x over previous
import functools

import jax
import jax.numpy as jnp
from jax import lax
from jax.experimental import pallas as pl
from jax.experimental.pallas import tpu as pltpu
from jax.experimental.pallas import tpu_sc as plsc

B, N, M, C1, C2 = 4, 8192, 2048, 64, 128
H = 128
EPS_W = 1e-8
EPS_BN = 1e-5
TILE = 512
NT = N // TILE
BIG = 3.0e38
F32 = jnp.float32
BF16 = jnp.bfloat16

BN_ = B * N
NW = 32
PPW = N // NW
CH = 128
NCH = PPW // CH


def _p1_body(u_ref, k_ref, kt_ref, idx_ref, w_ref):
    u = u_ref[0]
    k = k_ref[0]
    kt = kt_ref[0]
    u2 = jnp.sum(u * u, axis=1)
    k2 = (kt[0] * kt[0] + kt[1] * kt[1]) + kt[2] * kt[2]
    cross = jax.lax.dot_general(u.astype(BF16), k.astype(BF16),
                                (((1,), (1,)), ((), ())),
                                preferred_element_type=F32)
    d = u2[:, None] + k2[None, :] - 2.0 * cross

    iota = jax.lax.broadcasted_iota(jnp.int32, (TILE, M), 1)
    dc = d
    wun, idxs = [], []
    for kk in range(3):
        m = jnp.min(dc, axis=1, keepdims=True)
        i = jnp.min(jnp.where(dc == m, iota, M), axis=1, keepdims=True)
        wun.append(1.0 / (m + EPS_W))
        idxs.append(i)
        if kk < 2:
            dc = jnp.where(iota == i, BIG, dc)

    wsum = (wun[0] + wun[2]) + wun[1]
    for kk in range(3):
        idx_ref[kk] = jnp.transpose(idxs[kk], (1, 0))
        w_ref[kk] = jnp.transpose(wun[kk] / wsum, (1, 0))


def _sc_interp(kf_hbm, idx_hbm, w_hbm, out_hbm,
               i0, i1, i2, w0, w1, w2, r0, r1, r2, acc, s0, s1, s2):
    wid = lax.axis_index("s") * 2 + lax.axis_index("c")
    base0 = wid * PPW

    def chunk(c, carry):
        base = base0 + c * CH
        pltpu.sync_copy(idx_hbm.at[0, 0, pl.ds(base, CH)], i0)
        pltpu.sync_copy(idx_hbm.at[1, 0, pl.ds(base, CH)], i1)
        pltpu.sync_copy(idx_hbm.at[2, 0, pl.ds(base, CH)], i2)
        a0 = pltpu.async_copy(kf_hbm.at[i0], r0, s0)
        a1 = pltpu.async_copy(kf_hbm.at[i1], r1, s1)
        a2 = pltpu.async_copy(kf_hbm.at[i2], r2, s2)
        pltpu.sync_copy(w_hbm.at[0, 0, pl.ds(base, CH)], w0)
        pltpu.sync_copy(w_hbm.at[1, 0, pl.ds(base, CH)], w1)
        pltpu.sync_copy(w_hbm.at[2, 0, pl.ds(base, CH)], w2)
        a0.wait()
        a1.wait()
        a2.wait()

        def group(g, carry2):
            wv0 = w0[pl.ds(g * 16, 16)]
            wv1 = w1[pl.ds(g * 16, 16)]
            wv2 = w2[pl.ds(g * 16, 16)]
            for j in range(16):
                p = g * 16 + j
                ws0 = wv0[j]
                ws1 = wv1[j]
                ws2 = wv2[j]
                for v in range(H // 16):
                    sl = pl.ds(v * 16, 16)
                    acc[p, sl] = (r0[p, sl] * ws0 + r1[p, sl] * ws1
                                  + r2[p, sl] * ws2)
            return carry2

        lax.fori_loop(0, CH // 16, group, 0)
        pltpu.sync_copy(acc, out_hbm.at[pl.ds(base, CH)])
        return carry

    lax.fori_loop(0, NCH, chunk, 0)


def _p2_body(uf_ref, in_ref, w0a_ref, w0b_ref, h0_ref, s0_ref, q0_ref):
    first = jnp.logical_and(pl.program_id(0) == 0, pl.program_id(1) == 0)
    h0 = (jax.lax.dot_general(uf_ref[0].astype(BF16), w0a_ref[...],
                              (((0,), (0,)), ((), ())),
                              preferred_element_type=F32)
          + jnp.dot(in_ref[0].astype(BF16), w0b_ref[...],
                    preferred_element_type=F32))
    h0_ref[0] = h0

    @pl.when(first)
    def _():
        s0_ref[...] = jnp.zeros_like(s0_ref)
        q0_ref[...] = jnp.zeros_like(q0_ref)

    s0_ref[...] += jnp.sum(h0, axis=0, keepdims=True)
    q0_ref[...] += jnp.sum(h0 * h0, axis=0, keepdims=True)


def _p3_body(h0_ref, sc_ref, sh_ref, w1_ref, h1_ref, s1_ref, q1_ref):
    first = jnp.logical_and(pl.program_id(0) == 0, pl.program_id(1) == 0)
    a = jnp.maximum(h0_ref[0] * sc_ref[...] + sh_ref[...], 0.0)
    h1 = jnp.dot(a.astype(BF16), w1_ref[...], preferred_element_type=F32)
    h1_ref[0] = h1

    @pl.when(first)
    def _():
        s1_ref[...] = jnp.zeros_like(s1_ref)
        q1_ref[...] = jnp.zeros_like(q1_ref)

    s1_ref[...] += jnp.sum(h1, axis=0, keepdims=True)
    q1_ref[...] += jnp.sum(h1 * h1, axis=0, keepdims=True)


def _p4_body(h1_ref, sc_ref, sh_ref, out_ref):
    y = jnp.maximum(h1_ref[0] * sc_ref[...] + sh_ref[...], 0.0)
    out_ref[0] = y.T


def _affine(s, q, gamma, beta):
    n = float(B * N)
    mean = s / n
    var = q / n - mean * mean
    scale = gamma[None, :] / jnp.sqrt(var + EPS_BN)
    shift = beta[None, :] - mean * scale
    return scale, shift


def kernel(unknown, known, unknow_feats, known_feats, W0, g0, b0, W1, g1, b1):
    k_t = jnp.transpose(known, (0, 2, 1))
    kf = jnp.transpose(known_feats, (0, 2, 1))
    w0a = W0[:C1].astype(BF16)
    w0b = W0[C1:].astype(BF16)
    w1b = W1.astype(BF16)

    sc_fn = functools.partial(
        pl.kernel,
        out_type=jax.ShapeDtypeStruct((N, C2), F32),
        mesh=plsc.VectorSubcoreMesh(core_axis_name="c", subcore_axis_name="s"),
        scratch_types=[
            pltpu.VMEM((CH,), jnp.int32),
            pltpu.VMEM((CH,), jnp.int32),
            pltpu.VMEM((CH,), jnp.int32),
            pltpu.VMEM((CH,), F32),
            pltpu.VMEM((CH,), F32),
            pltpu.VMEM((CH,), F32),
            pltpu.VMEM((CH, C2), F32),
            pltpu.VMEM((CH, C2), F32),
            pltpu.VMEM((CH, C2), F32),
            pltpu.VMEM((CH, C2), F32),
            pltpu.SemaphoreType.DMA,
            pltpu.SemaphoreType.DMA,
            pltpu.SemaphoreType.DMA,
        ],
    )(_sc_interp)

    interp_parts = []
    for b in range(B):
        idxg, wn = pl.pallas_call(
            _p1_body,
            grid=(NT,),
            in_specs=[
                pl.BlockSpec((1, TILE, 3), lambda t, b=b: (b, t, 0)),
                pl.BlockSpec((1, M, 3), lambda t, b=b: (b, 0, 0)),
                pl.BlockSpec((1, 3, M), lambda t, b=b: (b, 0, 0)),
            ],
            out_specs=[
                pl.BlockSpec((3, 1, TILE), lambda t: (0, 0, t)),
                pl.BlockSpec((3, 1, TILE), lambda t: (0, 0, t)),
            ],
            out_shape=[
                jax.ShapeDtypeStruct((3, 1, N), jnp.int32),
                jax.ShapeDtypeStruct((3, 1, N), F32),
            ],
            compiler_params=pltpu.CompilerParams(
                dimension_semantics=("arbitrary",)),
        )(unknown, known, k_t)
        interp_parts.append(sc_fn(kf[b], idxg, wn))
    interp = jnp.stack(interp_parts)

    h0, s0, q0 = pl.pallas_call(
        _p2_body,
        grid=(B, NT),
        in_specs=[
            pl.BlockSpec((1, C1, TILE), lambda b, t: (b, 0, t)),
            pl.BlockSpec((1, TILE, C2), lambda b, t: (b, t, 0)),
            pl.BlockSpec((C1, H), lambda b, t: (0, 0)),
            pl.BlockSpec((C2, H), lambda b, t: (0, 0)),
        ],
        out_specs=[
            pl.BlockSpec((1, TILE, H), lambda b, t: (b, t, 0)),
            pl.BlockSpec((1, H), lambda b, t: (0, 0)),
            pl.BlockSpec((1, H), lambda b, t: (0, 0)),
        ],
        out_shape=[
            jax.ShapeDtypeStruct((B, N, H), F32),
            jax.ShapeDtypeStruct((1, H), F32),
            jax.ShapeDtypeStruct((1, H), F32),
        ],
        compiler_params=pltpu.CompilerParams(
            dimension_semantics=("arbitrary", "arbitrary")),
    )(unknow_feats, interp, w0a, w0b)

    sc0, sh0 = _affine(s0, q0, g0, b0)

    h1, s1, q1 = pl.pallas_call(
        _p3_body,
        grid=(B, NT),
        in_specs=[
            pl.BlockSpec((1, TILE, H), lambda b, t: (b, t, 0)),
            pl.BlockSpec((1, H), lambda b, t: (0, 0)),
            pl.BlockSpec((1, H), lambda b, t: (0, 0)),
            pl.BlockSpec((H, H), lambda b, t: (0, 0)),
        ],
        out_specs=[
            pl.BlockSpec((1, TILE, H), lambda b, t: (b, t, 0)),
            pl.BlockSpec((1, H), lambda b, t: (0, 0)),
            pl.BlockSpec((1, H), lambda b, t: (0, 0)),
        ],
        out_shape=[
            jax.ShapeDtypeStruct((B, N, H), F32),
            jax.ShapeDtypeStruct((1, H), F32),
            jax.ShapeDtypeStruct((1, H), F32),
        ],
        compiler_params=pltpu.CompilerParams(
            dimension_semantics=("arbitrary", "arbitrary")),
    )(h0, sc0, sh0, w1b)

    sc1, sh1 = _affine(s1, q1, g1, b1)

    out = pl.pallas_call(
        _p4_body,
        grid=(B, NT),
        in_specs=[
            pl.BlockSpec((1, TILE, H), lambda b, t: (b, t, 0)),
            pl.BlockSpec((1, H), lambda b, t: (0, 0)),
            pl.BlockSpec((1, H), lambda b, t: (0, 0)),
        ],
        out_specs=pl.BlockSpec((1, H, TILE), lambda b, t: (b, 0, t)),
        out_shape=jax.ShapeDtypeStruct((B, H, N), F32),
        compiler_params=pltpu.CompilerParams(
            dimension_semantics=("arbitrary", "arbitrary")),
    )(h1, sc1, sh1)

    return out


# speedup vs baseline: 41.5099x; 1.1351x over previous
import functools

import jax
import jax.numpy as jnp
from jax import lax
from jax.experimental import pallas as pl
from jax.experimental.pallas import tpu as pltpu
from jax.experimental.pallas import tpu_sc as plsc

B, N, M, C1, C2 = 4, 8192, 2048, 64, 128
H = 128
EPS_W = 1e-8
EPS_BN = 1e-5
TILE = 1024
NT = N // TILE
BIG = 3.0e38
F32 = jnp.float32
BF16 = jnp.bfloat16

BN_ = B * N
NW = 32
PPW = N // NW
CH = 128
NCH = PPW // CH


def _p1_body(u_ref, k_ref, kt_ref, idx_ref, w_ref):
    u = u_ref[0]
    k = k_ref[0]
    kt = kt_ref[0]
    u2 = jnp.sum(u * u, axis=1)
    k2 = (kt[0] * kt[0] + kt[1] * kt[1]) + kt[2] * kt[2]
    cross = jax.lax.dot_general(u.astype(BF16), k.astype(BF16),
                                (((1,), (1,)), ((), ())),
                                preferred_element_type=F32)
    d = u2[:, None] + k2[None, :] - 2.0 * cross

    iota = jax.lax.broadcasted_iota(jnp.int32, (TILE, M), 1)
    dc = d
    wun, idxs = [], []
    for kk in range(3):
        m = jnp.min(dc, axis=1, keepdims=True)
        i = jnp.min(jnp.where(dc == m, iota, M), axis=1, keepdims=True)
        wun.append(1.0 / (m + EPS_W))
        idxs.append(i)
        if kk < 2:
            dc = jnp.where(iota == i, BIG, dc)

    wsum = (wun[0] + wun[2]) + wun[1]
    for kk in range(3):
        idx_ref[kk] = jnp.transpose(idxs[kk], (1, 0))
        w_ref[kk] = jnp.transpose(wun[kk] / wsum, (1, 0))


def _sc_interp(kf_hbm, idx_hbm, w_hbm, out_hbm,
               i0, i1, i2, w0, w1, w2, r0, r1, r2, acc, s0, s1, s2):
    wid = lax.axis_index("s") * 2 + lax.axis_index("c")
    base0 = wid * PPW

    def chunk(c, carry):
        base = base0 + c * CH
        pltpu.sync_copy(idx_hbm.at[0, 0, pl.ds(base, CH)], i0)
        pltpu.sync_copy(idx_hbm.at[1, 0, pl.ds(base, CH)], i1)
        pltpu.sync_copy(idx_hbm.at[2, 0, pl.ds(base, CH)], i2)
        a0 = pltpu.async_copy(kf_hbm.at[i0], r0, s0)
        a1 = pltpu.async_copy(kf_hbm.at[i1], r1, s1)
        a2 = pltpu.async_copy(kf_hbm.at[i2], r2, s2)
        pltpu.sync_copy(w_hbm.at[0, 0, pl.ds(base, CH)], w0)
        pltpu.sync_copy(w_hbm.at[1, 0, pl.ds(base, CH)], w1)
        pltpu.sync_copy(w_hbm.at[2, 0, pl.ds(base, CH)], w2)
        a0.wait()
        a1.wait()
        a2.wait()

        def group(g, carry2):
            wv0 = w0[pl.ds(g * 16, 16)]
            wv1 = w1[pl.ds(g * 16, 16)]
            wv2 = w2[pl.ds(g * 16, 16)]
            for j in range(16):
                p = g * 16 + j
                ws0 = wv0[j]
                ws1 = wv1[j]
                ws2 = wv2[j]
                for v in range(H // 16):
                    sl = pl.ds(v * 16, 16)
                    acc[p, sl] = (r0[p, sl] * ws0 + r1[p, sl] * ws1
                                  + r2[p, sl] * ws2)
            return carry2

        lax.fori_loop(0, CH // 16, group, 0)
        pltpu.sync_copy(acc, out_hbm.at[pl.ds(base, CH)])
        return carry

    lax.fori_loop(0, NCH, chunk, 0)


def _p2_body(uf_ref, in_ref, w0a_ref, w0b_ref, h0_ref, s0_ref, q0_ref):
    first = jnp.logical_and(pl.program_id(0) == 0, pl.program_id(1) == 0)
    h0 = (jax.lax.dot_general(uf_ref[0].astype(BF16), w0a_ref[...],
                              (((0,), (0,)), ((), ())),
                              preferred_element_type=F32)
          + jnp.dot(in_ref[0].astype(BF16), w0b_ref[...],
                    preferred_element_type=F32))
    h0_ref[0] = h0

    @pl.when(first)
    def _():
        s0_ref[...] = jnp.zeros_like(s0_ref)
        q0_ref[...] = jnp.zeros_like(q0_ref)

    s0_ref[...] += jnp.sum(h0, axis=0, keepdims=True)
    q0_ref[...] += jnp.sum(h0 * h0, axis=0, keepdims=True)


def _p3_body(h0_ref, sc_ref, sh_ref, w1_ref, h1_ref, s1_ref, q1_ref):
    first = jnp.logical_and(pl.program_id(0) == 0, pl.program_id(1) == 0)
    a = jnp.maximum(h0_ref[0] * sc_ref[...] + sh_ref[...], 0.0)
    h1 = jnp.dot(a.astype(BF16), w1_ref[...], preferred_element_type=F32)
    h1_ref[0] = h1

    @pl.when(first)
    def _():
        s1_ref[...] = jnp.zeros_like(s1_ref)
        q1_ref[...] = jnp.zeros_like(q1_ref)

    s1_ref[...] += jnp.sum(h1, axis=0, keepdims=True)
    q1_ref[...] += jnp.sum(h1 * h1, axis=0, keepdims=True)


def _p4_body(h1_ref, sc_ref, sh_ref, out_ref):
    y = jnp.maximum(h1_ref[0] * sc_ref[...] + sh_ref[...], 0.0)
    out_ref[0] = y.T


def _affine(s, q, gamma, beta):
    n = float(B * N)
    mean = s / n
    var = q / n - mean * mean
    scale = gamma[None, :] / jnp.sqrt(var + EPS_BN)
    shift = beta[None, :] - mean * scale
    return scale, shift


def kernel(unknown, known, unknow_feats, known_feats, W0, g0, b0, W1, g1, b1):
    k_t = jnp.transpose(known, (0, 2, 1))
    kf = jnp.transpose(known_feats, (0, 2, 1))
    w0a = W0[:C1].astype(BF16)
    w0b = W0[C1:].astype(BF16)
    w1b = W1.astype(BF16)

    sc_fn = functools.partial(
        pl.kernel,
        out_type=jax.ShapeDtypeStruct((N, C2), F32),
        mesh=plsc.VectorSubcoreMesh(core_axis_name="c", subcore_axis_name="s"),
        scratch_types=[
            pltpu.VMEM((CH,), jnp.int32),
            pltpu.VMEM((CH,), jnp.int32),
            pltpu.VMEM((CH,), jnp.int32),
            pltpu.VMEM((CH,), F32),
            pltpu.VMEM((CH,), F32),
            pltpu.VMEM((CH,), F32),
            pltpu.VMEM((CH, C2), F32),
            pltpu.VMEM((CH, C2), F32),
            pltpu.VMEM((CH, C2), F32),
            pltpu.VMEM((CH, C2), F32),
            pltpu.SemaphoreType.DMA,
            pltpu.SemaphoreType.DMA,
            pltpu.SemaphoreType.DMA,
        ],
    )(_sc_interp)

    interp_parts = []
    for b in range(B):
        idxg, wn = pl.pallas_call(
            _p1_body,
            grid=(NT,),
            in_specs=[
                pl.BlockSpec((1, TILE, 3), lambda t, b=b: (b, t, 0)),
                pl.BlockSpec((1, M, 3), lambda t, b=b: (b, 0, 0)),
                pl.BlockSpec((1, 3, M), lambda t, b=b: (b, 0, 0)),
            ],
            out_specs=[
                pl.BlockSpec((3, 1, TILE), lambda t: (0, 0, t)),
                pl.BlockSpec((3, 1, TILE), lambda t: (0, 0, t)),
            ],
            out_shape=[
                jax.ShapeDtypeStruct((3, 1, N), jnp.int32),
                jax.ShapeDtypeStruct((3, 1, N), F32),
            ],
            compiler_params=pltpu.CompilerParams(
                dimension_semantics=("arbitrary",)),
        )(unknown, known, k_t)
        interp_parts.append(sc_fn(kf[b], idxg, wn))
    interp = jnp.stack(interp_parts)

    h0, s0, q0 = pl.pallas_call(
        _p2_body,
        grid=(B, NT),
        in_specs=[
            pl.BlockSpec((1, C1, TILE), lambda b, t: (b, 0, t)),
            pl.BlockSpec((1, TILE, C2), lambda b, t: (b, t, 0)),
            pl.BlockSpec((C1, H), lambda b, t: (0, 0)),
            pl.BlockSpec((C2, H), lambda b, t: (0, 0)),
        ],
        out_specs=[
            pl.BlockSpec((1, TILE, H), lambda b, t: (b, t, 0)),
            pl.BlockSpec((1, H), lambda b, t: (0, 0)),
            pl.BlockSpec((1, H), lambda b, t: (0, 0)),
        ],
        out_shape=[
            jax.ShapeDtypeStruct((B, N, H), F32),
            jax.ShapeDtypeStruct((1, H), F32),
            jax.ShapeDtypeStruct((1, H), F32),
        ],
        compiler_params=pltpu.CompilerParams(
            dimension_semantics=("arbitrary", "arbitrary")),
    )(unknow_feats, interp, w0a, w0b)

    sc0, sh0 = _affine(s0, q0, g0, b0)

    h1, s1, q1 = pl.pallas_call(
        _p3_body,
        grid=(B, NT),
        in_specs=[
            pl.BlockSpec((1, TILE, H), lambda b, t: (b, t, 0)),
            pl.BlockSpec((1, H), lambda b, t: (0, 0)),
            pl.BlockSpec((1, H), lambda b, t: (0, 0)),
            pl.BlockSpec((H, H), lambda b, t: (0, 0)),
        ],
        out_specs=[
            pl.BlockSpec((1, TILE, H), lambda b, t: (b, t, 0)),
            pl.BlockSpec((1, H), lambda b, t: (0, 0)),
            pl.BlockSpec((1, H), lambda b, t: (0, 0)),
        ],
        out_shape=[
            jax.ShapeDtypeStruct((B, N, H), F32),
            jax.ShapeDtypeStruct((1, H), F32),
            jax.ShapeDtypeStruct((1, H), F32),
        ],
        compiler_params=pltpu.CompilerParams(
            dimension_semantics=("arbitrary", "arbitrary")),
    )(h0, sc0, sh0, w1b)

    sc1, sh1 = _affine(s1, q1, g1, b1)

    out = pl.pallas_call(
        _p4_body,
        grid=(B, NT),
        in_specs=[
            pl.BlockSpec((1, TILE, H), lambda b, t: (b, t, 0)),
            pl.BlockSpec((1, H), lambda b, t: (0, 0)),
            pl.BlockSpec((1, H), lambda b, t: (0, 0)),
        ],
        out_specs=pl.BlockSpec((1, H, TILE), lambda b, t: (b, 0, t)),
        out_shape=jax.ShapeDtypeStruct((B, H, N), F32),
        compiler_params=pltpu.CompilerParams(
            dimension_semantics=("arbitrary", "arbitrary")),
    )(h1, sc1, sh1)

    return out
